# baseline (speedup 1.0000x reference)
import jax
import jax.numpy as jnp
from jax.experimental import pallas as pl
from jax.experimental.pallas import tpu as pltpu

_SLICES = 2


def _prep_kernel(adj_ref, an_ref):
    a = adj_ref[...]
    n = a.shape[0]
    row = jax.lax.broadcasted_iota(jnp.int32, (n, n), 0)
    col = jax.lax.broadcasted_iota(jnp.int32, (n, n), 1)
    a = a + jnp.where(row == col, 1.0, 0.0).astype(a.dtype)
    d = jnp.sum(a, axis=1, keepdims=True)
    u = jnp.full((1, n), 1.0, jnp.float32)
    d_row = jax.lax.dot_general(
        u, a, (((1,), (1,)), ((), ())),
        precision=jax.lax.Precision.HIGHEST,
        preferred_element_type=jnp.float32)
    an_ref[...] = a * jax.lax.rsqrt(d) * jax.lax.rsqrt(d_row)


def _layernorm(h, ones_m, g, be, eps=1e-5):
    mu = jnp.dot(h, ones_m, preferred_element_type=jnp.float32)
    c = h - mu
    var = jnp.dot(c * c, ones_m, preferred_element_type=jnp.float32)
    return c * jax.lax.rsqrt(var + eps) * g + be


def _gcn_kernel(an_ref, x_ref,
                w1_ref, b1_ref, g1_ref, be1_ref,
                w2_ref, b2_ref, g2_ref, be2_ref,
                wo_ref, bo_ref, go_ref, beo_ref, o_ref):
    an = an_ref[...]
    f = x_ref.shape[-1]
    ones_m = jnp.full((f, f), 1.0 / f, jnp.float32)
    for s in range(_SLICES):
        xx = x_ref[0, s]
        h = xx
        for (w_r, b_r, g_r, be_r) in ((w1_ref, b1_ref, g1_ref, be1_ref),
                                      (w2_ref, b2_ref, g2_ref, be2_ref)):
            hn = _layernorm(h, ones_m, g_r[...], be_r[...])
            m = jnp.dot(an, hn, preferred_element_type=jnp.float32)
            h = h + jax.nn.relu(
                jnp.dot(m, w_r[...], preferred_element_type=jnp.float32)
                + b_r[...])
        ho = _layernorm(h, ones_m, go_ref[...], beo_ref[...])
        o_ref[0, s] = (jnp.dot(ho, wo_ref[...],
                            preferred_element_type=jnp.float32)
                    + bo_ref[...] + xx)


def kernel(x, adj, W1, b1, g1, be1, W2, b2, g2, be2, Wo, bo, go, beo):
    B, T, N, F = x.shape
    BT = B * T
    S = _SLICES
    xr = x.reshape(BT // S, S, N, F)

    an = pl.pallas_call(
        _prep_kernel,
        out_shape=jax.ShapeDtypeStruct((N, N), jnp.float32),
    )(adj)

    vec = lambda v: v.reshape(1, -1)
    full = lambda shp: pl.BlockSpec(shp, lambda i: (0,) * len(shp))
    out = pl.pallas_call(
        _gcn_kernel,
        grid=(BT // S,),
        in_specs=[
            full((N, N)),
            pl.BlockSpec((1, S, N, F), lambda i: (i, 0, 0, 0)),
            full((128, 128)), full((1, 128)), full((1, 128)), full((1, 128)),
            full((128, 128)), full((1, 128)), full((1, 128)), full((1, 128)),
            full((128, 128)), full((1, 128)), full((1, 128)), full((1, 128)),
        ],
        out_specs=pl.BlockSpec((1, S, N, F), lambda i: (i, 0, 0, 0)),
        out_shape=jax.ShapeDtypeStruct((BT // S, S, N, F), jnp.float32),
        compiler_params=pltpu.CompilerParams(
            dimension_semantics=("parallel",)),
    )(an, xr, W1, vec(b1), vec(g1), vec(be1),
      W2, vec(b2), vec(g2), vec(be2),
      Wo, vec(bo), vec(go), vec(beo))
    return out.reshape(B, T, N, F)


# speedup vs baseline: 1.0255x; 1.0255x over previous
import jax
import jax.numpy as jnp
from jax.experimental import pallas as pl
from jax.experimental.pallas import tpu as pltpu

_SLICES = 4


def _prep_kernel(adj_ref, an_ref):
    a = adj_ref[...]
    n = a.shape[0]
    row = jax.lax.broadcasted_iota(jnp.int32, (n, n), 0)
    col = jax.lax.broadcasted_iota(jnp.int32, (n, n), 1)
    a = a + jnp.where(row == col, 1.0, 0.0).astype(a.dtype)
    d = jnp.sum(a, axis=1, keepdims=True)
    u = jnp.full((1, n), 1.0, jnp.float32)
    d_row = jax.lax.dot_general(
        u, a, (((1,), (1,)), ((), ())),
        precision=jax.lax.Precision.HIGHEST,
        preferred_element_type=jnp.float32)
    an_ref[...] = a * jax.lax.rsqrt(d) * jax.lax.rsqrt(d_row)


def _layernorm(h, ones_m, g, be, eps=1e-5):
    mu = jnp.dot(h, ones_m, preferred_element_type=jnp.float32)
    c = h - mu
    var = jnp.dot(c * c, ones_m, preferred_element_type=jnp.float32)
    return c * jax.lax.rsqrt(var + eps) * g + be


def _gcn_kernel(an_ref, x_ref,
                w1_ref, b1_ref, g1_ref, be1_ref,
                w2_ref, b2_ref, g2_ref, be2_ref,
                wo_ref, bo_ref, go_ref, beo_ref, o_ref):
    an = an_ref[...]
    f = x_ref.shape[-1]
    ones_m = jnp.full((f, f), 1.0 / f, jnp.float32)
    for s in range(_SLICES):
        xx = x_ref[0, s]
        h = xx
        for (w_r, b_r, g_r, be_r) in ((w1_ref, b1_ref, g1_ref, be1_ref),
                                      (w2_ref, b2_ref, g2_ref, be2_ref)):
            hn = _layernorm(h, ones_m, g_r[...], be_r[...])
            m = jnp.dot(an, hn, preferred_element_type=jnp.float32)
            h = h + jax.nn.relu(
                jnp.dot(m, w_r[...], preferred_element_type=jnp.float32)
                + b_r[...])
        ho = _layernorm(h, ones_m, go_ref[...], beo_ref[...])
        o_ref[0, s] = (jnp.dot(ho, wo_ref[...],
                            preferred_element_type=jnp.float32)
                    + bo_ref[...] + xx)


def kernel(x, adj, W1, b1, g1, be1, W2, b2, g2, be2, Wo, bo, go, beo):
    B, T, N, F = x.shape
    BT = B * T
    S = _SLICES
    xr = x.reshape(BT // S, S, N, F)

    an = pl.pallas_call(
        _prep_kernel,
        out_shape=jax.ShapeDtypeStruct((N, N), jnp.float32),
    )(adj)

    vec = lambda v: v.reshape(1, -1)
    full = lambda shp: pl.BlockSpec(shp, lambda i: (0,) * len(shp))
    out = pl.pallas_call(
        _gcn_kernel,
        grid=(BT // S,),
        in_specs=[
            full((N, N)),
            pl.BlockSpec((1, S, N, F), lambda i: (i, 0, 0, 0)),
            full((128, 128)), full((1, 128)), full((1, 128)), full((1, 128)),
            full((128, 128)), full((1, 128)), full((1, 128)), full((1, 128)),
            full((128, 128)), full((1, 128)), full((1, 128)), full((1, 128)),
        ],
        out_specs=pl.BlockSpec((1, S, N, F), lambda i: (i, 0, 0, 0)),
        out_shape=jax.ShapeDtypeStruct((BT // S, S, N, F), jnp.float32),
        compiler_params=pltpu.CompilerParams(
            dimension_semantics=("parallel",)),
    )(an, xr, W1, vec(b1), vec(g1), vec(be1),
      W2, vec(b2), vec(g2), vec(be2),
      Wo, vec(bo), vec(go), vec(beo))
    return out.reshape(B, T, N, F)


# speedup vs baseline: 1.2978x; 1.2655x over previous
import jax
import jax.numpy as jnp
from jax.experimental import pallas as pl
from jax.experimental.pallas import tpu as pltpu

_SLICES = 4


def _prep_kernel(adj_ref, an_ref):
    a = adj_ref[...]
    n = a.shape[0]
    row = jax.lax.broadcasted_iota(jnp.int32, (n, n), 0)
    col = jax.lax.broadcasted_iota(jnp.int32, (n, n), 1)
    a = a + jnp.where(row == col, 1.0, 0.0).astype(a.dtype)
    d = jnp.sum(a, axis=1, keepdims=True)
    u = jnp.full((1, n), 1.0, jnp.float32)
    d_row = jax.lax.dot_general(
        u, a, (((1,), (1,)), ((), ())),
        precision=jax.lax.Precision.HIGHEST,
        preferred_element_type=jnp.float32)
    an = a * jax.lax.rsqrt(d) * jax.lax.rsqrt(d_row)
    an_ref[...] = an.astype(jnp.bfloat16)


def _layernorm(h, g, be, eps=1e-5):
    mu = jnp.mean(h, axis=-1, keepdims=True)
    c = h - mu
    var = jnp.mean(c * c, axis=-1, keepdims=True)
    return c * jax.lax.rsqrt(var + eps) * g + be


def _gcn_kernel(an_ref, x_ref,
                w1_ref, b1_ref, g1_ref, be1_ref,
                w2_ref, b2_ref, g2_ref, be2_ref,
                wo_ref, bo_ref, go_ref, beo_ref, o_ref):
    an = an_ref[...]
    for s in range(_SLICES):
        xx = x_ref[0, s]
        h = xx
        for (w_r, b_r, g_r, be_r) in ((w1_ref, b1_ref, g1_ref, be1_ref),
                                      (w2_ref, b2_ref, g2_ref, be2_ref)):
            hn = _layernorm(h, g_r[...], be_r[...]).astype(jnp.bfloat16)
            m = jnp.dot(an, hn, preferred_element_type=jnp.float32)
            h = h + jax.nn.relu(
                jnp.dot(m, w_r[...], preferred_element_type=jnp.float32)
                + b_r[...])
        ho = _layernorm(h, go_ref[...], beo_ref[...])
        o_ref[0, s] = (jnp.dot(ho, wo_ref[...],
                               preferred_element_type=jnp.float32)
                       + bo_ref[...] + xx)


def kernel(x, adj, W1, b1, g1, be1, W2, b2, g2, be2, Wo, bo, go, beo):
    B, T, N, F = x.shape
    BT = B * T
    S = _SLICES
    xr = x.reshape(BT // S, S, N, F)

    an = pl.pallas_call(
        _prep_kernel,
        out_shape=jax.ShapeDtypeStruct((N, N), jnp.bfloat16),
    )(adj)

    vec = lambda v: v.reshape(1, -1)
    full = lambda shp: pl.BlockSpec(shp, lambda i: (0,) * len(shp))
    out = pl.pallas_call(
        _gcn_kernel,
        grid=(BT // S,),
        in_specs=[
            full((N, N)),
            pl.BlockSpec((1, S, N, F), lambda i: (i, 0, 0, 0)),
            full((128, 128)), full((1, 128)), full((1, 128)), full((1, 128)),
            full((128, 128)), full((1, 128)), full((1, 128)), full((1, 128)),
            full((128, 128)), full((1, 128)), full((1, 128)), full((1, 128)),
        ],
        out_specs=pl.BlockSpec((1, S, N, F), lambda i: (i, 0, 0, 0)),
        out_shape=jax.ShapeDtypeStruct((BT // S, S, N, F), jnp.float32),
        compiler_params=pltpu.CompilerParams(
            dimension_semantics=("parallel",)),
    )(an, xr, W1, vec(b1), vec(g1), vec(be1),
      W2, vec(b2), vec(g2), vec(be2),
      Wo, vec(bo), vec(go), vec(beo))
    return out.reshape(B, T, N, F)


# speedup vs baseline: 1.3520x; 1.0417x over previous
import jax
import jax.numpy as jnp
from jax.experimental import pallas as pl
from jax.experimental.pallas import tpu as pltpu

_SLICES = 8


def _prep_kernel(adj_ref, an_ref):
    a = adj_ref[...]
    n = a.shape[0]
    row = jax.lax.broadcasted_iota(jnp.int32, (n, n), 0)
    col = jax.lax.broadcasted_iota(jnp.int32, (n, n), 1)
    a = a + jnp.where(row == col, 1.0, 0.0).astype(a.dtype)
    d = jnp.sum(a, axis=1, keepdims=True)
    u = jnp.full((1, n), 1.0, jnp.float32)
    d_row = jax.lax.dot_general(
        u, a, (((1,), (1,)), ((), ())),
        precision=jax.lax.Precision.HIGHEST,
        preferred_element_type=jnp.float32)
    an = a * jax.lax.rsqrt(d) * jax.lax.rsqrt(d_row)
    an_ref[...] = an.astype(jnp.bfloat16)


def _layernorm(h, g, be, eps=1e-5):
    mu = jnp.mean(h, axis=-1, keepdims=True)
    c = h - mu
    var = jnp.mean(c * c, axis=-1, keepdims=True)
    return c * jax.lax.rsqrt(var + eps) * g + be


def _gcn_kernel(an_ref, x_ref,
                w1_ref, b1_ref, g1_ref, be1_ref,
                w2_ref, b2_ref, g2_ref, be2_ref,
                wo_ref, bo_ref, go_ref, beo_ref, o_ref):
    an = an_ref[...]
    for s in range(_SLICES):
        xx = x_ref[0, s]
        h = xx
        for (w_r, b_r, g_r, be_r) in ((w1_ref, b1_ref, g1_ref, be1_ref),
                                      (w2_ref, b2_ref, g2_ref, be2_ref)):
            hn = _layernorm(h, g_r[...], be_r[...]).astype(jnp.bfloat16)
            m = jnp.dot(an, hn, preferred_element_type=jnp.float32)
            h = h + jax.nn.relu(
                jnp.dot(m, w_r[...], preferred_element_type=jnp.float32)
                + b_r[...])
        ho = _layernorm(h, go_ref[...], beo_ref[...])
        o_ref[0, s] = (jnp.dot(ho, wo_ref[...],
                               preferred_element_type=jnp.float32)
                       + bo_ref[...] + xx)


def kernel(x, adj, W1, b1, g1, be1, W2, b2, g2, be2, Wo, bo, go, beo):
    B, T, N, F = x.shape
    BT = B * T
    S = _SLICES
    xr = x.reshape(BT // S, S, N, F)

    an = pl.pallas_call(
        _prep_kernel,
        out_shape=jax.ShapeDtypeStruct((N, N), jnp.bfloat16),
    )(adj)

    vec = lambda v: v.reshape(1, -1)
    full = lambda shp: pl.BlockSpec(shp, lambda i: (0,) * len(shp))
    out = pl.pallas_call(
        _gcn_kernel,
        grid=(BT // S,),
        in_specs=[
            full((N, N)),
            pl.BlockSpec((1, S, N, F), lambda i: (i, 0, 0, 0)),
            full((128, 128)), full((1, 128)), full((1, 128)), full((1, 128)),
            full((128, 128)), full((1, 128)), full((1, 128)), full((1, 128)),
            full((128, 128)), full((1, 128)), full((1, 128)), full((1, 128)),
        ],
        out_specs=pl.BlockSpec((1, S, N, F), lambda i: (i, 0, 0, 0)),
        out_shape=jax.ShapeDtypeStruct((BT // S, S, N, F), jnp.float32),
        compiler_params=pltpu.CompilerParams(
            dimension_semantics=("parallel",)),
    )(an, xr, W1, vec(b1), vec(g1), vec(be1),
      W2, vec(b2), vec(g2), vec(be2),
      Wo, vec(bo), vec(go), vec(beo))
    return out.reshape(B, T, N, F)


# speedup vs baseline: 1.3648x; 1.0095x over previous
import jax
import jax.numpy as jnp
from jax.experimental import pallas as pl
from jax.experimental.pallas import tpu as pltpu

_SLICES = 12


def _prep_kernel(adj_ref, an_ref):
    a = adj_ref[...]
    n = a.shape[0]
    row = jax.lax.broadcasted_iota(jnp.int32, (n, n), 0)
    col = jax.lax.broadcasted_iota(jnp.int32, (n, n), 1)
    a = a + jnp.where(row == col, 1.0, 0.0).astype(a.dtype)
    d = jnp.sum(a, axis=1, keepdims=True)
    u = jnp.full((1, n), 1.0, jnp.float32)
    d_row = jax.lax.dot_general(
        u, a, (((1,), (1,)), ((), ())),
        precision=jax.lax.Precision.HIGHEST,
        preferred_element_type=jnp.float32)
    an = a * jax.lax.rsqrt(d) * jax.lax.rsqrt(d_row)
    an_ref[...] = an.astype(jnp.bfloat16)


def _layernorm(h, g, be, eps=1e-5):
    mu = jnp.mean(h, axis=-1, keepdims=True)
    c = h - mu
    var = jnp.mean(c * c, axis=-1, keepdims=True)
    return c * jax.lax.rsqrt(var + eps) * g + be


def _gcn_kernel(an_ref, x_ref,
                w1_ref, b1_ref, g1_ref, be1_ref,
                w2_ref, b2_ref, g2_ref, be2_ref,
                wo_ref, bo_ref, go_ref, beo_ref, o_ref):
    an = an_ref[...]
    for s in range(_SLICES):
        xx = x_ref[0, s]
        h = xx
        for (w_r, b_r, g_r, be_r) in ((w1_ref, b1_ref, g1_ref, be1_ref),
                                      (w2_ref, b2_ref, g2_ref, be2_ref)):
            hn = _layernorm(h, g_r[...], be_r[...]).astype(jnp.bfloat16)
            m = jnp.dot(an, hn, preferred_element_type=jnp.float32)
            h = h + jax.nn.relu(
                jnp.dot(m, w_r[...], preferred_element_type=jnp.float32)
                + b_r[...])
        ho = _layernorm(h, go_ref[...], beo_ref[...])
        o_ref[0, s] = (jnp.dot(ho, wo_ref[...],
                               preferred_element_type=jnp.float32)
                       + bo_ref[...] + xx)


def kernel(x, adj, W1, b1, g1, be1, W2, b2, g2, be2, Wo, bo, go, beo):
    B, T, N, F = x.shape
    BT = B * T
    S = _SLICES
    xr = x.reshape(BT // S, S, N, F)

    an = pl.pallas_call(
        _prep_kernel,
        out_shape=jax.ShapeDtypeStruct((N, N), jnp.bfloat16),
    )(adj)

    vec = lambda v: v.reshape(1, -1)
    full = lambda shp: pl.BlockSpec(shp, lambda i: (0,) * len(shp))
    out = pl.pallas_call(
        _gcn_kernel,
        grid=(BT // S,),
        in_specs=[
            full((N, N)),
            pl.BlockSpec((1, S, N, F), lambda i: (i, 0, 0, 0)),
            full((128, 128)), full((1, 128)), full((1, 128)), full((1, 128)),
            full((128, 128)), full((1, 128)), full((1, 128)), full((1, 128)),
            full((128, 128)), full((1, 128)), full((1, 128)), full((1, 128)),
        ],
        out_specs=pl.BlockSpec((1, S, N, F), lambda i: (i, 0, 0, 0)),
        out_shape=jax.ShapeDtypeStruct((BT // S, S, N, F), jnp.float32),
        compiler_params=pltpu.CompilerParams(
            dimension_semantics=("parallel",)),
    )(an, xr, W1, vec(b1), vec(g1), vec(be1),
      W2, vec(b2), vec(g2), vec(be2),
      Wo, vec(bo), vec(go), vec(beo))
    return out.reshape(B, T, N, F)


# speedup vs baseline: 1.3695x; 1.0035x over previous
import jax
import jax.numpy as jnp
from jax.experimental import pallas as pl
from jax.experimental.pallas import tpu as pltpu

_SLICES = 12


def _col(v_row, eye):
    return jax.lax.dot_general(
        eye, v_row, (((1,), (1,)), ((), ())),
        precision=jax.lax.Precision.HIGHEST,
        preferred_element_type=jnp.float32)


def _prep_kernel(adj_ref, w1_ref, b1_ref, g1_ref, be1_ref,
                 w2_ref, b2_ref, g2_ref, be2_ref,
                 wo_ref, bo_ref, go_ref, beo_ref,
                 an_ref, wg1_ref, wg2_ref, wgo_ref,
                 bias1_ref, bias2_ref, biaso_ref):
    a = adj_ref[...]
    n = a.shape[0]
    row = jax.lax.broadcasted_iota(jnp.int32, (n, n), 0)
    col = jax.lax.broadcasted_iota(jnp.int32, (n, n), 1)
    a = a + jnp.where(row == col, 1.0, 0.0).astype(a.dtype)
    d = jnp.sum(a, axis=1, keepdims=True)
    u = jnp.full((1, n), 1.0, jnp.float32)
    d_row = jax.lax.dot_general(
        u, a, (((1,), (1,)), ((), ())),
        precision=jax.lax.Precision.HIGHEST,
        preferred_element_type=jnp.float32)
    an = a * jax.lax.rsqrt(d) * jax.lax.rsqrt(d_row)
    an_ref[...] = an.astype(jnp.bfloat16)
    rs = jnp.sum(an, axis=1, keepdims=True)

    f = w1_ref.shape[0]
    eye = jnp.where(
        jax.lax.broadcasted_iota(jnp.int32, (f, f), 0)
        == jax.lax.broadcasted_iota(jnp.int32, (f, f), 1), 1.0, 0.0)
    hp = jax.lax.Precision.HIGHEST
    for (w_r, b_r, g_r, be_r, wg_r, bias_r) in (
            (w1_ref, b1_ref, g1_ref, be1_ref, wg1_ref, bias1_ref),
            (w2_ref, b2_ref, g2_ref, be2_ref, wg2_ref, bias2_ref)):
        w = w_r[...]
        wg_r[...] = _col(g_r[...], eye) * w
        bew = jnp.dot(be_r[...], w, precision=hp,
                      preferred_element_type=jnp.float32)
        bias_r[...] = rs * bew + b_r[...]
    wo = wo_ref[...]
    wgo_ref[...] = _col(go_ref[...], eye) * wo
    biaso_ref[...] = jnp.dot(beo_ref[...], wo, precision=hp,
                             preferred_element_type=jnp.float32) + bo_ref[...]


def _ln_core(h, eps=1e-5):
    mu = jnp.mean(h, axis=-1, keepdims=True)
    c = h - mu
    var = jnp.mean(c * c, axis=-1, keepdims=True)
    return c * jax.lax.rsqrt(var + eps)


def _gcn_kernel(an_ref, x_ref, wg1_ref, wg2_ref, wgo_ref,
                bias1_ref, bias2_ref, biaso_ref, o_ref):
    an = an_ref[...]
    for s in range(_SLICES):
        xx = x_ref[0, s]
        h = xx
        for (wg_r, bias_r) in ((wg1_ref, bias1_ref), (wg2_ref, bias2_ref)):
            z = _ln_core(h).astype(jnp.bfloat16)
            mz = jnp.dot(an, z, preferred_element_type=jnp.float32)
            h = h + jax.nn.relu(
                jnp.dot(mz, wg_r[...], preferred_element_type=jnp.float32)
                + bias_r[...])
        zo = _ln_core(h)
        o_ref[0, s] = (jnp.dot(zo, wgo_ref[...],
                               preferred_element_type=jnp.float32)
                       + biaso_ref[...] + xx)


def kernel(x, adj, W1, b1, g1, be1, W2, b2, g2, be2, Wo, bo, go, beo):
    B, T, N, F = x.shape
    BT = B * T
    S = _SLICES
    xr = x.reshape(BT // S, S, N, F)
    vec = lambda v: v.reshape(1, -1)

    f32 = jnp.float32
    an, wg1, wg2, wgo, bias1, bias2, biaso = pl.pallas_call(
        _prep_kernel,
        out_shape=[jax.ShapeDtypeStruct((N, N), jnp.bfloat16),
                   jax.ShapeDtypeStruct((F, F), f32),
                   jax.ShapeDtypeStruct((F, F), f32),
                   jax.ShapeDtypeStruct((F, F), f32),
                   jax.ShapeDtypeStruct((N, F), f32),
                   jax.ShapeDtypeStruct((N, F), f32),
                   jax.ShapeDtypeStruct((1, F), f32)],
    )(adj, W1, vec(b1), vec(g1), vec(be1),
      W2, vec(b2), vec(g2), vec(be2),
      Wo, vec(bo), vec(go), vec(beo))

    full = lambda shp: pl.BlockSpec(shp, lambda i: (0,) * len(shp))
    out = pl.pallas_call(
        _gcn_kernel,
        grid=(BT // S,),
        in_specs=[
            full((N, N)),
            pl.BlockSpec((1, S, N, F), lambda i: (i, 0, 0, 0)),
            full((F, F)), full((F, F)), full((F, F)),
            full((N, F)), full((N, F)), full((1, F)),
        ],
        out_specs=pl.BlockSpec((1, S, N, F), lambda i: (i, 0, 0, 0)),
        out_shape=jax.ShapeDtypeStruct((BT // S, S, N, F), jnp.float32),
        compiler_params=pltpu.CompilerParams(
            dimension_semantics=("parallel",)),
    )(an, xr, wg1, wg2, wgo, bias1, bias2, biaso)
    return out.reshape(B, T, N, F)
